# baseline (speedup 1.0000x reference)
import functools

import jax
import jax.numpy as jnp
from jax.experimental import pallas as pl
from jax.experimental.pallas import tpu as pltpu


def _body(x_ref, mt_ref, w_ref, bv_ref, fc_ref, out_ref, pt_ref, qt_ref,
          t_ref, g_ref, *, kk, filt, nmxu, nfp, nfr, tb):
    nvpu = filt - nmxu

    @pl.when(pl.program_id(0) == 0)
    def _build_ops():
        mv = mt_ref[...]
        r = jax.lax.broadcasted_iota(jnp.int32, (nfp, nfp), 0)
        c = jax.lax.broadcasted_iota(jnp.int32, (nfp, nfp), 1)
        t0 = (r == c).astype(jnp.float32)
        pt_ref[0, :, :] = t0.astype(jnp.bfloat16)
        pt_ref[1, :, :] = mv.astype(jnp.bfloat16)
        t1 = mv
        for k in range(2, kk):
            t2 = 2.0 * jax.lax.dot(mv, t1, precision=jax.lax.Precision.HIGHEST,
                                   preferred_element_type=jnp.float32) - t0
            pt_ref[k, :, :] = t2.astype(jnp.bfloat16)
            t0, t1 = t1, t2

        def qbody(f, carry):
            acc = pt_ref[0, :, :].astype(jnp.float32) * w_ref[0, f]
            for k in range(1, kk):
                acc = acc + pt_ref[k, :, :].astype(jnp.float32) * w_ref[k, f]
            qt_ref[f, :, :] = acc.astype(jnp.bfloat16)
            return carry

        jax.lax.fori_loop(0, nmxu, qbody, 0)

    xb = x_ref[...].astype(jnp.bfloat16)
    t_ref[0, :, :] = x_ref[...]
    for k in range(1, kk):
        t_ref[k, :, :] = jax.lax.dot(pt_ref[k, :, :], xb,
                                     preferred_element_type=jnp.float32)

    def fbody(j, carry):
        gm = jax.lax.dot(qt_ref[j, :, :], xb,
                         preferred_element_type=jnp.float32)
        gm = jnp.maximum(gm[:nfr, :] + bv_ref[j], 0.0)
        g_ref[pl.ds(j * nfr, nfr), :] = gm.astype(jnp.bfloat16)
        for s in range(nvpu // nmxu):
            fv = nmxu + s * nmxu + j
            acc = t_ref[0, :nfr, :] * w_ref[0, fv]
            for k in range(1, kk):
                acc = acc + t_ref[k, :nfr, :] * w_ref[k, fv]
            gv = jnp.maximum(acc + bv_ref[fv], 0.0)
            g_ref[pl.ds(fv * nfr, nfr), :] = gv.astype(jnp.bfloat16)
        return carry

    jax.lax.fori_loop(0, nmxu, fbody, 0)
    out_ref[...] = jax.lax.dot(fc_ref[...], g_ref[...],
                               preferred_element_type=jnp.float32)


def kernel(x, L, W, b, fc_w, fc_b, y):
    B, N, F = x.shape
    K, FILT = W.shape
    C = fc_w.shape[1]
    NF = N * F
    NFP = 384
    NFR = 320
    TB = 512
    NMXU = 32
    GROWS = FILT * NFR

    xT = jnp.pad(x.reshape(B, NF).T, ((0, NFP - NF), (0, 0)))
    Mt = jnp.kron(L, jnp.eye(F, dtype=L.dtype))
    Mtp = jnp.pad(Mt, ((0, NFP - NF), (0, NFP - NF)))
    bvec = b.reshape(FILT)
    fc3 = jnp.pad(fc_w.reshape(FILT, NF, C), ((0, 0), (0, NFR - NF), (0, 0)))
    fcT = jnp.pad(fc3.transpose(2, 0, 1).reshape(C, GROWS),
                  ((0, 8 - C), (0, 0))).astype(jnp.bfloat16)

    body = functools.partial(_body, kk=K, filt=FILT, nmxu=NMXU, nfp=NFP,
                             nfr=NFR, tb=TB)
    out = pl.pallas_call(
        body,
        grid=(B // TB,),
        in_specs=[
            pl.BlockSpec((NFP, TB), lambda i: (0, i)),
            pl.BlockSpec((NFP, NFP), lambda i: (0, 0)),
            pl.BlockSpec(memory_space=pltpu.SMEM),
            pl.BlockSpec(memory_space=pltpu.SMEM),
            pl.BlockSpec((8, GROWS), lambda i: (0, 0)),
        ],
        out_specs=pl.BlockSpec((8, TB), lambda i: (0, i)),
        out_shape=jax.ShapeDtypeStruct((8, B), jnp.float32),
        scratch_shapes=[
            pltpu.VMEM((K, NFP, NFP), jnp.bfloat16),
            pltpu.VMEM((NMXU, NFP, NFP), jnp.bfloat16),
            pltpu.VMEM((K, NFP, TB), jnp.float32),
            pltpu.VMEM((GROWS, TB), jnp.bfloat16),
        ],
        compiler_params=pltpu.CompilerParams(
            dimension_semantics=("arbitrary",)),
    )(xT, Mtp, W, bvec, fcT)
    return out[:C, :].T + fc_b[None, :]


# speedup vs baseline: 1.3785x; 1.3785x over previous
import functools

import jax
import jax.numpy as jnp
from jax.experimental import pallas as pl
from jax.experimental.pallas import tpu as pltpu


def _body(x_ref, mt_ref, w_ref, bv_ref, fc_ref, out_ref, pt_ref, t_ref,
          g_ref, *, kk, filt, nfp, nfr, tb, fpair):
    @pl.when(pl.program_id(0) == 0)
    def _build_pt():
        mv = mt_ref[...]
        r = jax.lax.broadcasted_iota(jnp.int32, (nfp, nfp), 0)
        c = jax.lax.broadcasted_iota(jnp.int32, (nfp, nfp), 1)
        t0 = (r == c).astype(jnp.float32)
        pt_ref[0, :, :] = t0.astype(jnp.bfloat16)
        pt_ref[1, :, :] = mv.astype(jnp.bfloat16)
        t1 = mv
        for k in range(2, kk):
            t2 = 2.0 * jax.lax.dot(mv, t1, precision=jax.lax.Precision.HIGHEST,
                                   preferred_element_type=jnp.float32) - t0
            pt_ref[k, :, :] = t2.astype(jnp.bfloat16)
            t0, t1 = t1, t2

    xb = x_ref[...].astype(jnp.bfloat16)
    t_ref[0, :, :] = x_ref[...]
    for k in range(1, kk):
        t_ref[k, :, :] = jax.lax.dot(pt_ref[k, :, :], xb,
                                     preferred_element_type=jnp.float32)

    def fbody(j, carry):
        ts = [t_ref[k, :nfr, :] for k in range(kk)]
        for s in range(fpair):
            f = j * fpair + s
            acc = ts[0] * w_ref[0, f]
            for k in range(1, kk):
                acc = acc + ts[k] * w_ref[k, f]
            gv = jnp.maximum(acc + bv_ref[f], 0.0)
            g_ref[pl.ds(f * nfr, nfr), :] = gv.astype(jnp.bfloat16)
        return carry

    jax.lax.fori_loop(0, filt // fpair, fbody, 0)
    out_ref[...] = jax.lax.dot(fc_ref[...], g_ref[...],
                               preferred_element_type=jnp.float32)


def kernel(x, L, W, b, fc_w, fc_b, y):
    B, N, F = x.shape
    K, FILT = W.shape
    C = fc_w.shape[1]
    NF = N * F
    NFP = 384
    NFR = 320
    TB = 512
    FPAIR = 2
    GROWS = FILT * NFR

    xT = jnp.pad(x.reshape(B, NF).T, ((0, NFP - NF), (0, 0)))
    Mt = jnp.kron(L, jnp.eye(F, dtype=L.dtype))
    Mtp = jnp.pad(Mt, ((0, NFP - NF), (0, NFP - NF)))
    bvec = b.reshape(FILT)
    fc3 = jnp.pad(fc_w.reshape(FILT, NF, C), ((0, 0), (0, NFR - NF), (0, 0)))
    fcT = jnp.pad(fc3.transpose(2, 0, 1).reshape(C, GROWS),
                  ((0, 8 - C), (0, 0))).astype(jnp.bfloat16)

    body = functools.partial(_body, kk=K, filt=FILT, nfp=NFP, nfr=NFR, tb=TB,
                             fpair=FPAIR)
    out = pl.pallas_call(
        body,
        grid=(B // TB,),
        in_specs=[
            pl.BlockSpec((NFP, TB), lambda i: (0, i)),
            pl.BlockSpec((NFP, NFP), lambda i: (0, 0)),
            pl.BlockSpec(memory_space=pltpu.SMEM),
            pl.BlockSpec(memory_space=pltpu.SMEM),
            pl.BlockSpec((8, GROWS), lambda i: (0, 0)),
        ],
        out_specs=pl.BlockSpec((8, TB), lambda i: (0, i)),
        out_shape=jax.ShapeDtypeStruct((8, B), jnp.float32),
        scratch_shapes=[
            pltpu.VMEM((K, NFP, NFP), jnp.bfloat16),
            pltpu.VMEM((K, NFP, TB), jnp.float32),
            pltpu.VMEM((GROWS, TB), jnp.bfloat16),
        ],
        compiler_params=pltpu.CompilerParams(
            dimension_semantics=("arbitrary",)),
    )(xT, Mtp, W, bvec, fcT)
    return out[:C, :].T + fc_b[None, :]


# speedup vs baseline: 2.1493x; 1.5592x over previous
import functools

import jax
import jax.numpy as jnp
from jax.experimental import pallas as pl
from jax.experimental.pallas import tpu as pltpu


def _body(x_ref, mt_ref, w_ref, bv_ref, fc_ref, out_ref, pt_ref, qt_ref, *,
          kk, filt, nfp, nfr, tb, nchunk):
    grows = filt * nfr

    @pl.when(pl.program_id(0) == 0)
    def _build_ops():
        mv = mt_ref[...]
        r = jax.lax.broadcasted_iota(jnp.int32, (nfp, nfp), 0)
        c = jax.lax.broadcasted_iota(jnp.int32, (nfp, nfp), 1)
        t0 = (r == c).astype(jnp.float32)
        pt_ref[0, :, :] = t0
        pt_ref[1, :, :] = mv
        t1 = mv
        for k in range(2, kk):
            t2 = 2.0 * jax.lax.dot(mv, t1, precision=jax.lax.Precision.HIGHEST,
                                   preferred_element_type=jnp.float32) - t0
            pt_ref[k, :, :] = t2
            t0, t1 = t1, t2

        bmask = (jax.lax.broadcasted_iota(jnp.int32, (nfr, nfp), 1)
                 == nfp - 1).astype(jnp.float32)

        def qbody(f, carry):
            acc = pt_ref[0, :nfr, :] * w_ref[0, f]
            for k in range(1, kk):
                acc = acc + pt_ref[k, :nfr, :] * w_ref[k, f]
            acc = acc + bmask * bv_ref[f]
            qt_ref[pl.ds(f * nfr, nfr), :] = acc.astype(jnp.bfloat16)
            return carry

        jax.lax.fori_loop(0, filt, qbody, 0)

    xb = x_ref[...].astype(jnp.bfloat16)
    crows = grows // nchunk
    acc = jnp.zeros((8, tb), jnp.float32)
    for ch in range(nchunk):
        lo = ch * crows
        gch = jax.lax.dot(qt_ref[lo:lo + crows, :], xb,
                          preferred_element_type=jnp.float32)
        gch = jnp.maximum(gch, 0.0).astype(jnp.bfloat16)
        acc = acc + jax.lax.dot(fc_ref[:, lo:lo + crows], gch,
                                preferred_element_type=jnp.float32)
    out_ref[...] = acc


def kernel(x, L, W, b, fc_w, fc_b, y):
    B, N, F = x.shape
    K, FILT = W.shape
    C = fc_w.shape[1]
    NF = N * F
    NFP = 384
    NFR = 320
    TB = 512
    NCHUNK = 8
    GROWS = FILT * NFR

    xT = jnp.pad(x.reshape(B, NF).T, ((0, NFP - NF), (0, 0)))
    xT = xT.at[NFP - 1, :].set(1.0)
    Mt = jnp.kron(L, jnp.eye(F, dtype=L.dtype))
    Mtp = jnp.pad(Mt, ((0, NFP - NF), (0, NFP - NF)))
    bvec = b.reshape(FILT)
    fc3 = jnp.pad(fc_w.reshape(FILT, NF, C), ((0, 0), (0, NFR - NF), (0, 0)))
    fcT = jnp.pad(fc3.transpose(2, 0, 1).reshape(C, GROWS),
                  ((0, 8 - C), (0, 0))).astype(jnp.bfloat16)

    body = functools.partial(_body, kk=K, filt=FILT, nfp=NFP, nfr=NFR, tb=TB,
                             nchunk=NCHUNK)
    out = pl.pallas_call(
        body,
        grid=(B // TB,),
        in_specs=[
            pl.BlockSpec((NFP, TB), lambda i: (0, i)),
            pl.BlockSpec((NFP, NFP), lambda i: (0, 0)),
            pl.BlockSpec(memory_space=pltpu.SMEM),
            pl.BlockSpec(memory_space=pltpu.SMEM),
            pl.BlockSpec((8, GROWS), lambda i: (0, 0)),
        ],
        out_specs=pl.BlockSpec((8, TB), lambda i: (0, i)),
        out_shape=jax.ShapeDtypeStruct((8, B), jnp.float32),
        scratch_shapes=[
            pltpu.VMEM((K, NFP, NFP), jnp.float32),
            pltpu.VMEM((GROWS, NFP), jnp.bfloat16),
        ],
        compiler_params=pltpu.CompilerParams(
            dimension_semantics=("arbitrary",)),
    )(xT, Mtp, W, bvec, fcT)
    return out[:C, :].T + fc_b[None, :]


# speedup vs baseline: 3.8507x; 1.7916x over previous
import functools

import jax
import jax.numpy as jnp
from jax.experimental import pallas as pl
from jax.experimental.pallas import tpu as pltpu


def _body(x_ref, l_ref, w_ref, bv_ref, fc_ref, out_ref, p_ref, a_ref, *,
          kk, filt, feat, np_, tb):
    @pl.when(pl.program_id(0) == 0)
    def _build_ops():
        lv = l_ref[...]
        r = jax.lax.broadcasted_iota(jnp.int32, (np_, np_), 0)
        c = jax.lax.broadcasted_iota(jnp.int32, (np_, np_), 1)
        t0 = (r == c).astype(jnp.float32)
        p_ref[0, :, :] = t0
        p_ref[1, :, :] = lv
        t1 = lv
        for k in range(2, kk):
            t2 = 2.0 * jax.lax.dot(lv, t1, precision=jax.lax.Precision.HIGHEST,
                                   preferred_element_type=jnp.float32) - t0
            p_ref[k, :, :] = t2
            t0, t1 = t1, t2

        bmask = (jax.lax.broadcasted_iota(jnp.int32, (np_, np_), 1)
                 == np_ - 1).astype(jnp.float32)

        def abody(f, carry):
            acc = p_ref[0, :, :] * w_ref[0, f]
            for k in range(1, kk):
                acc = acc + p_ref[k, :, :] * w_ref[k, f]
            acc = acc + bmask * bv_ref[f]
            a_ref[pl.ds(f * np_, np_), :] = acc.astype(jnp.bfloat16)
            return carry

        jax.lax.fori_loop(0, filt, abody, 0)

    av = a_ref[...]
    acc = jnp.zeros((8, tb), jnp.float32)
    for s in range(feat):
        xs = x_ref[s, :, :].astype(jnp.bfloat16)
        g = jax.lax.dot(av, xs, preferred_element_type=jnp.float32)
        g = jnp.maximum(g, 0.0).astype(jnp.bfloat16)
        acc = acc + jax.lax.dot(fc_ref[s, :, :], g,
                                preferred_element_type=jnp.float32)
    out_ref[...] = acc


def kernel(x, L, W, b, fc_w, fc_b, y):
    B, N, F = x.shape
    K, FILT = W.shape
    C = fc_w.shape[1]
    NP = 64
    TB = 512
    AROWS = FILT * NP

    x5 = jnp.pad(x.transpose(2, 1, 0), ((0, 0), (0, NP - N), (0, 0)))
    x5 = x5.at[:, NP - 1, :].set(1.0)
    Lp = jnp.pad(L, ((0, NP - N), (0, NP - N)))
    bvec = b.reshape(FILT)
    fc5 = jnp.pad(fc_w.reshape(FILT, N, F, C).transpose(2, 3, 0, 1),
                  ((0, 0), (0, 8 - C), (0, 0), (0, NP - N)))
    fcT = fc5.reshape(F, 8, AROWS).astype(jnp.bfloat16)

    body = functools.partial(_body, kk=K, filt=FILT, feat=F, np_=NP, tb=TB)
    out = pl.pallas_call(
        body,
        grid=(B // TB,),
        in_specs=[
            pl.BlockSpec((F, NP, TB), lambda i: (0, 0, i)),
            pl.BlockSpec((NP, NP), lambda i: (0, 0)),
            pl.BlockSpec(memory_space=pltpu.SMEM),
            pl.BlockSpec(memory_space=pltpu.SMEM),
            pl.BlockSpec((F, 8, AROWS), lambda i: (0, 0, 0)),
        ],
        out_specs=pl.BlockSpec((8, TB), lambda i: (0, i)),
        out_shape=jax.ShapeDtypeStruct((8, B), jnp.float32),
        scratch_shapes=[
            pltpu.VMEM((K, NP, NP), jnp.float32),
            pltpu.VMEM((AROWS, NP), jnp.bfloat16),
        ],
        compiler_params=pltpu.CompilerParams(
            dimension_semantics=("arbitrary",)),
    )(x5, Lp, W, bvec, fcT)
    return out[:C, :].T + fc_b[None, :]


# speedup vs baseline: 3.8700x; 1.0050x over previous
import functools

import jax
import jax.numpy as jnp
from jax.experimental import pallas as pl
from jax.experimental.pallas import tpu as pltpu


def _body(x_ref, l_ref, w_ref, bv_ref, fc_ref, out_ref, p_ref, a_ref, *,
          kk, filt, feat, np_, tb):
    @pl.when(pl.program_id(0) == 0)
    def _build_ops():
        lv = l_ref[...]
        r = jax.lax.broadcasted_iota(jnp.int32, (np_, np_), 0)
        c = jax.lax.broadcasted_iota(jnp.int32, (np_, np_), 1)
        t0 = (r == c).astype(jnp.float32)
        p_ref[0, :, :] = t0
        p_ref[1, :, :] = lv
        t1 = lv
        for k in range(2, kk):
            t2 = 2.0 * jax.lax.dot(lv, t1, precision=jax.lax.Precision.HIGHEST,
                                   preferred_element_type=jnp.float32) - t0
            p_ref[k, :, :] = t2
            t0, t1 = t1, t2

        bmask = (jax.lax.broadcasted_iota(jnp.int32, (np_, np_), 1)
                 == np_ - 1).astype(jnp.float32)

        def abody(f, carry):
            acc = p_ref[0, :, :] * w_ref[0, f]
            for k in range(1, kk):
                acc = acc + p_ref[k, :, :] * w_ref[k, f]
            acc = acc + bmask * bv_ref[f]
            a_ref[pl.ds(f * np_, np_), :] = acc.astype(jnp.bfloat16)
            return carry

        jax.lax.fori_loop(0, filt, abody, 0)

    av = a_ref[...]
    acc = jnp.zeros((8, tb), jnp.float32)
    for s in range(feat):
        xs = x_ref[s, :, :].astype(jnp.bfloat16)
        g = jax.lax.dot(av, xs, preferred_element_type=jnp.float32)
        g = jnp.maximum(g, 0.0).astype(jnp.bfloat16)
        acc = acc + jax.lax.dot(fc_ref[s, :, :], g,
                                preferred_element_type=jnp.float32)
    out_ref[...] = acc


def kernel(x, L, W, b, fc_w, fc_b, y):
    B, N, F = x.shape
    K, FILT = W.shape
    C = fc_w.shape[1]
    NP = 64
    TB = 1024
    AROWS = FILT * NP

    x5 = jnp.pad(x.transpose(2, 1, 0), ((0, 0), (0, NP - N), (0, 0)))
    x5 = x5.at[:, NP - 1, :].set(1.0)
    Lp = jnp.pad(L, ((0, NP - N), (0, NP - N)))
    bvec = b.reshape(FILT)
    fc5 = jnp.pad(fc_w.reshape(FILT, N, F, C).transpose(2, 3, 0, 1),
                  ((0, 0), (0, 8 - C), (0, 0), (0, NP - N)))
    fcT = fc5.reshape(F, 8, AROWS).astype(jnp.bfloat16)

    body = functools.partial(_body, kk=K, filt=FILT, feat=F, np_=NP, tb=TB)
    out = pl.pallas_call(
        body,
        grid=(B // TB,),
        in_specs=[
            pl.BlockSpec((F, NP, TB), lambda i: (0, 0, i)),
            pl.BlockSpec((NP, NP), lambda i: (0, 0)),
            pl.BlockSpec(memory_space=pltpu.SMEM),
            pl.BlockSpec(memory_space=pltpu.SMEM),
            pl.BlockSpec((F, 8, AROWS), lambda i: (0, 0, 0)),
        ],
        out_specs=pl.BlockSpec((8, TB), lambda i: (0, i)),
        out_shape=jax.ShapeDtypeStruct((8, B), jnp.float32),
        scratch_shapes=[
            pltpu.VMEM((K, NP, NP), jnp.float32),
            pltpu.VMEM((AROWS, NP), jnp.bfloat16),
        ],
        compiler_params=pltpu.CompilerParams(
            dimension_semantics=("arbitrary",)),
    )(x5, Lp, W, bvec, fcT)
    return out[:C, :].T + fc_b[None, :]
